# baseline (speedup 1.0000x reference)
import functools

import jax
import jax.numpy as jnp
from jax import lax
from jax.experimental import pallas as pl
from jax.experimental.pallas import tpu as pltpu
from jax.experimental.pallas import tpu_sc as plsc

_K = 1024
_COMMITMENT_COST = 0.25


def _argmin_chain(zb, cb):
    scores = jax.lax.dot_general(
        cb, zb, (((1,), (0,)), ((), ())),
        preferred_element_type=jnp.float32)
    c2 = jnp.sum(cb * cb, axis=1, keepdims=True)
    z2 = jnp.sum(zb * zb, axis=0, keepdims=True)
    dists = (z2 - 2.0 * scores) + c2

    minv = jnp.min(dists, axis=0, keepdims=True)
    rows = jax.lax.broadcasted_iota(jnp.int32, dists.shape, 0)
    idx = jnp.min(jnp.where(dists == minv, rows, _K),
                  axis=0, keepdims=True)
    return idx, jnp.sum(minv)


def _argmin_kernel(n_split, z_ref, cb_ref, idx_ref, sse_ref):
    cb = cb_ref[...]
    ts = z_ref.shape[2] // n_split
    tile_sse = 0.0
    for i in range(n_split):
        sl = pl.ds(i * ts, ts)
        idx, sse = _argmin_chain(z_ref[0, :, sl], cb)
        idx_ref[0, 0, :, sl] = idx
        tile_sse += sse

    @pl.when(jnp.logical_and(pl.program_id(0) == 0, pl.program_id(1) == 0))
    def _init():
        sse_ref[0, 0] = 0.0

    sse_ref[0, 0] += tile_sse


def _make_sc_gather(n, d, n_chunks):
    info = plsc.get_sparse_core_info()
    nw = info.num_cores * info.num_subcores
    b_per_w = n // nw
    bc = b_per_w // n_chunks
    mesh = plsc.VectorSubcoreMesh(core_axis_name="c", subcore_axis_name="s")

    @functools.partial(
        pl.kernel, mesh=mesh,
        out_type=jax.ShapeDtypeStruct((n, d), jnp.float32),
        scratch_types=[
            pltpu.VMEM((bc,), jnp.int32),
            pltpu.VMEM((bc, d), jnp.float32),
            pltpu.SemaphoreType.DMA,
        ],
    )
    def gather_k(table_hbm, idx_hbm, out_hbm, idx_v, rows_v, sem):
        wid = lax.axis_index("s") * info.num_cores + lax.axis_index("c")
        base = wid * b_per_w
        for c in range(n_chunks):
            off = base + c * bc
            pltpu.sync_copy(idx_hbm.at[pl.ds(off, bc)], idx_v)
            pltpu.async_copy(table_hbm.at[idx_v], rows_v, sem).wait()
            pltpu.sync_copy(rows_v, out_hbm.at[pl.ds(off, bc)])

    return gather_k


@functools.partial(jax.jit, static_argnames=("tile_s", "n_split"))
def _vq(z, codebook, tile_s=2048, n_split=2):
    B, D, d, h, w = z.shape
    S = d * h * w
    N = B * S
    ns = S // tile_s
    zr = z.reshape(B, D, S)

    idx, sse = pl.pallas_call(
        functools.partial(_argmin_kernel, n_split),
        grid=(B, ns),
        in_specs=[
            pl.BlockSpec((1, D, tile_s), lambda b, s: (b, 0, s)),
            pl.BlockSpec((_K, D), lambda b, s: (0, 0)),
        ],
        out_specs=[
            pl.BlockSpec((1, 1, 1, tile_s), lambda b, s: (b, s, 0, 0)),
            pl.BlockSpec(memory_space=pltpu.SMEM),
        ],
        out_shape=[
            jax.ShapeDtypeStruct((B, ns, 1, tile_s), jnp.int32),
            jax.ShapeDtypeStruct((1, 1), jnp.float32),
        ],
    )(zr, codebook)

    idx_flat = idx.reshape(N)
    cb_pad = jnp.pad(codebook, ((0, 0), (0, 128 - D)))
    zq_rows = _make_sc_gather(N, 128, 4)(cb_pad, idx_flat)

    zq = zq_rows.reshape(B, d, h, w, 128)[..., :D].transpose(0, 4, 1, 2, 3)
    loss = sse[0, 0] * (1.0 + _COMMITMENT_COST) / z.size
    return (zq, loss, idx.reshape(B, d, h, w))


def kernel(z, codebook):
    return _vq(z, codebook)


# speedup vs baseline: 1.2247x; 1.2247x over previous
import functools

import jax
import jax.numpy as jnp
from jax.experimental import pallas as pl
from jax.experimental.pallas import tpu as pltpu

_K = 1024
_COMMITMENT_COST = 0.25


def _vq_chain(zb, cb, n_chunks):
    scores = jax.lax.dot_general(
        cb, zb, (((1,), (0,)), ((), ())),
        preferred_element_type=jnp.float32)
    c2 = jnp.sum(cb * cb, axis=1, keepdims=True)
    z2 = jnp.sum(zb * zb, axis=0, keepdims=True)

    ck = _K // n_chunks
    runmin = runidx = None
    for j in range(n_chunks):
        sl = slice(j * ck, (j + 1) * ck)
        d = (z2 - 2.0 * scores[sl, :]) + c2[sl, :]
        cmin = jnp.min(d, axis=0, keepdims=True)
        rows = jax.lax.broadcasted_iota(jnp.int32, d.shape, 0) + (j * ck)
        cidx = jnp.min(jnp.where(d == cmin, rows, _K),
                       axis=0, keepdims=True)
        if j == 0:
            runmin, runidx = cmin, cidx
        else:
            upd = cmin < runmin
            runmin = jnp.where(upd, cmin, runmin)
            runidx = jnp.where(upd, cidx, runidx)

    rows_full = jax.lax.broadcasted_iota(jnp.int32, (_K,) + zb.shape[1:], 0)
    onehot = (rows_full == runidx).astype(jnp.float32)

    zq = jax.lax.dot_general(
        cb, onehot, (((0,), (0,)), ((), ())),
        preferred_element_type=jnp.float32)

    return zq, runidx, jnp.sum(runmin)


def _vq_kernel(n_split, n_chunks, z_ref, cb_ref, zq_ref, idx_ref, sse_ref):
    cb = cb_ref[...]
    ts = z_ref.shape[2] // n_split

    tile_sse = 0.0
    for i in range(n_split):
        sl = pl.ds(i * ts, ts)
        zq, idx, sse = _vq_chain(z_ref[0, :, sl], cb, n_chunks)
        zq_ref[0, :, sl] = zq
        idx_ref[0, 0, :, sl] = idx
        tile_sse += sse

    @pl.when(jnp.logical_and(pl.program_id(0) == 0, pl.program_id(1) == 0))
    def _init():
        sse_ref[0, 0] = 0.0

    sse_ref[0, 0] += tile_sse


@functools.partial(jax.jit, static_argnames=("tile_s", "n_split", "n_chunks"))
def _vq(z, codebook, tile_s=4096, n_split=2, n_chunks=1):
    B, D, d, h, w = z.shape
    S = d * h * w
    ns = S // tile_s
    zr = z.reshape(B, D, S)

    zq, idx, sse = pl.pallas_call(
        functools.partial(_vq_kernel, n_split, n_chunks),
        grid=(B, ns),
        in_specs=[
            pl.BlockSpec((1, D, tile_s), lambda b, s: (b, 0, s)),
            pl.BlockSpec((_K, D), lambda b, s: (0, 0)),
        ],
        out_specs=[
            pl.BlockSpec((1, D, tile_s), lambda b, s: (b, 0, s)),
            pl.BlockSpec((1, 1, 1, tile_s), lambda b, s: (b, s, 0, 0)),
            pl.BlockSpec(memory_space=pltpu.SMEM),
        ],
        out_shape=[
            jax.ShapeDtypeStruct((B, D, S), jnp.float32),
            jax.ShapeDtypeStruct((B, ns, 1, tile_s), jnp.int32),
            jax.ShapeDtypeStruct((1, 1), jnp.float32),
        ],
    )(zr, codebook)

    loss = sse[0, 0] * (1.0 + _COMMITMENT_COST) / z.size
    return (zq.reshape(B, D, d, h, w), loss, idx.reshape(B, d, h, w))


def kernel(z, codebook):
    return _vq(z, codebook)


# speedup vs baseline: 1.2345x; 1.0080x over previous
import functools

import jax
import jax.numpy as jnp
from jax.experimental import pallas as pl
from jax.experimental.pallas import tpu as pltpu

_K = 1024
_COMMITMENT_COST = 0.25


def _vq_chain(zb, cb, n_chunks):
    scores = jax.lax.dot_general(
        cb, zb, (((1,), (0,)), ((), ())),
        preferred_element_type=jnp.float32)
    c2 = jnp.sum(cb * cb, axis=1, keepdims=True)
    z2 = jnp.sum(zb * zb, axis=0, keepdims=True)

    ck = _K // n_chunks
    runmin = runidx = None
    for j in range(n_chunks):
        sl = slice(j * ck, (j + 1) * ck)
        d = (z2 - 2.0 * scores[sl, :]) + c2[sl, :]
        cmin = jnp.min(d, axis=0, keepdims=True)
        rows = jax.lax.broadcasted_iota(jnp.int32, d.shape, 0) + (j * ck)
        cidx = jnp.min(jnp.where(d == cmin, rows, _K),
                       axis=0, keepdims=True)
        if j == 0:
            runmin, runidx = cmin, cidx
        else:
            upd = cmin < runmin
            runmin = jnp.where(upd, cmin, runmin)
            runidx = jnp.where(upd, cidx, runidx)

    rows_full = jax.lax.broadcasted_iota(jnp.int32, (_K,) + zb.shape[1:], 0)
    onehot = (rows_full == runidx).astype(jnp.float32)

    zq = jax.lax.dot_general(
        cb, onehot, (((0,), (0,)), ((), ())),
        preferred_element_type=jnp.float32)

    return zq, runidx, jnp.sum(runmin)


def _vq_kernel(n_split, n_chunks, z_ref, cb_ref, zq_ref, idx_ref, sse_ref):
    cb = cb_ref[...]
    ts = z_ref.shape[2] // n_split

    tile_sse = 0.0
    for i in range(n_split):
        sl = pl.ds(i * ts, ts)
        zq, idx, sse = _vq_chain(z_ref[0, :, sl], cb, n_chunks)
        zq_ref[0, :, sl] = zq
        idx_ref[0, 0, :, sl] = idx
        tile_sse += sse

    @pl.when(jnp.logical_and(pl.program_id(0) == 0, pl.program_id(1) == 0))
    def _init():
        sse_ref[0, 0] = 0.0

    sse_ref[0, 0] += tile_sse


@functools.partial(jax.jit, static_argnames=("tile_s", "n_split", "n_chunks"))
def _vq(z, codebook, tile_s=8192, n_split=4, n_chunks=1):
    B, D, d, h, w = z.shape
    S = d * h * w
    ns = S // tile_s
    zr = z.reshape(B, D, S)

    zq, idx, sse = pl.pallas_call(
        functools.partial(_vq_kernel, n_split, n_chunks),
        grid=(B, ns),
        in_specs=[
            pl.BlockSpec((1, D, tile_s), lambda b, s: (b, 0, s)),
            pl.BlockSpec((_K, D), lambda b, s: (0, 0)),
        ],
        out_specs=[
            pl.BlockSpec((1, D, tile_s), lambda b, s: (b, 0, s)),
            pl.BlockSpec((1, 1, 1, tile_s), lambda b, s: (b, s, 0, 0)),
            pl.BlockSpec(memory_space=pltpu.SMEM),
        ],
        out_shape=[
            jax.ShapeDtypeStruct((B, D, S), jnp.float32),
            jax.ShapeDtypeStruct((B, ns, 1, tile_s), jnp.int32),
            jax.ShapeDtypeStruct((1, 1), jnp.float32),
        ],
    )(zr, codebook)

    loss = sse[0, 0] * (1.0 + _COMMITMENT_COST) / z.size
    return (zq.reshape(B, D, d, h, w), loss, idx.reshape(B, d, h, w))


def kernel(z, codebook):
    return _vq(z, codebook)
